# baseline (speedup 1.0000x reference)
import functools

import jax
import jax.numpy as jnp
from jax import lax
from jax.experimental import pallas as pl
from jax.experimental.pallas import tpu as pltpu
from jax.experimental.pallas import tpu_sc as plsc

E = 320000
D = 128
PD = 64
N = 10000

NSEG = 2
ESEG = E // NSEG
NC = 2
NS = 16
NW = NC * NS
EPW = ESEG // NW
CHUNK = 40
NCHUNK = EPW // CHUNK
NBUF = 5
RPS = 624
TAIL = N - NS * RPS

GATE_B = 16000


def _gate_body(lp_ref, x_ref, wt_ref, out_ref):
    x = x_ref[...]
    a = x[:, :PD]
    b = x[:, PD:]
    wt = wt_ref[...]
    ga = jnp.clip(lax.dot(a, wt, preferred_element_type=jnp.float32), 0.0, 1.0)
    gb = jnp.clip(lax.dot(b, wt, preferred_element_type=jnp.float32), 0.0, 1.0)
    at = jnp.concatenate([a, a], axis=1)
    bt = jnp.concatenate([b, b], axis=1)
    lp = jnp.abs(lp_ref[0])
    comb = lp * (at * ga + bt * gb) + x
    lo16 = lax.bitcast_convert_type(comb[:, :PD].astype(jnp.bfloat16),
                                    jnp.uint16)
    hi16 = lax.bitcast_convert_type(comb[:, PD:].astype(jnp.bfloat16),
                                    jnp.uint16)
    out_ref[...] = (hi16.astype(jnp.int32) << 16) | lo16.astype(jnp.int32)


def _gate(x, wt, lp, seg):
    nblk = ESEG // GATE_B
    return pl.pallas_call(
        _gate_body,
        grid=(nblk,),
        in_specs=[
            pl.BlockSpec(memory_space=pltpu.SMEM),
            pl.BlockSpec((GATE_B, D), lambda i, _o=seg * nblk: (i + _o, 0)),
            pl.BlockSpec((PD, D), lambda i: (0, 0)),
        ],
        out_specs=pl.BlockSpec((GATE_B, PD), lambda i: (i, 0)),
        out_shape=jax.ShapeDtypeStruct((ESEG, PD), jnp.int32),
    )(lp, x, wt)


def _sc_scatter(comb, idx, init, seg):
    mesh = plsc.VectorSubcoreMesh(core_axis_name="c", subcore_axis_name="s")
    ibase0 = seg * ESEG

    @functools.partial(
        pl.kernel,
        mesh=mesh,
        out_type=jax.ShapeDtypeStruct((NC * N, D), jnp.float32),
        scratch_types=(
            [pltpu.VMEM((CHUNK * PD,), jnp.int32) for _ in range(NBUF)]
            + [pltpu.VMEM((CHUNK, D), jnp.float32) for _ in range(NBUF)]
            + [pltpu.VMEM((CHUNK,), jnp.int32) for _ in range(2 * NBUF)]
            + [pltpu.VMEM_SHARED((N, D), jnp.float32)]
            + [pltpu.SemaphoreType.DMA for _ in range(2 * NBUF)]
        ),
    )
    def run(comb_hbm, idx_hbm, init_hbm, out_hbm,
            eb0, eb1, eb2, eb3, eb4, fb0, fb1, fb2, fb3, fb4,
            ti0, ti1, ti2, ti3, ti4, ib0, ib1, ib2, ib3, ib4,
            acc, sg0, sg1, sg2, sg3, sg4, ss0, ss1, ss2, ss3, ss4):
        ebufs = [eb0, eb1, eb2, eb3, eb4]
        fbufs = [fb0, fb1, fb2, fb3, fb4]
        tibufs = [ti0, ti1, ti2, ti3, ti4]
        ibufs = [ib0, ib1, ib2, ib3, ib4]
        sgs = [sg0, sg1, sg2, sg3, sg4]
        sss = [ss0, ss1, ss2, ss3, ss4]
        c = lax.axis_index("c")
        s = lax.axis_index("s")
        wid = s * NC + c
        base = wid * EPW
        ibase = ibase0 + base

        for b in range(NBUF):
            off = b * CHUNK
            pltpu.async_copy(
                comb_hbm.at[pl.ds((base + off) * PD, CHUNK * PD)],
                ebufs[b], sgs[b])
            pltpu.async_copy(idx_hbm.at[pl.ds(ibase + off, CHUNK)],
                             tibufs[b], sgs[b])

        pltpu.sync_copy(init_hbm.at[pl.ds(c * N + s * RPS, RPS)],
                        acc.at[pl.ds(s * RPS, RPS)])

        @pl.when(s == 0)
        def _():
            pltpu.sync_copy(init_hbm.at[pl.ds(c * N + NS * RPS, TAIL)],
                            acc.at[pl.ds(NS * RPS, TAIL)])

        plsc.subcore_barrier()

        def body(g, carry):
            for b in range(NBUF):
                i = g * NBUF + b
                pltpu.make_async_copy(
                    comb_hbm.at[pl.ds(base * PD, CHUNK * PD)],
                    ebufs[b], sgs[b]).wait()
                pltpu.make_async_copy(
                    idx_hbm.at[pl.ds(base, CHUNK)], tibufs[b], sgs[b]).wait()

                @pl.when(i >= NBUF)
                def _():
                    pltpu.make_async_copy(
                        fbufs[b], acc.at[ibufs[b]], sss[b]).wait()

                for o in (0, 16, 24):
                    ibufs[b][pl.ds(o, 16)] = tibufs[b][pl.ds(o, 16)]

                def conv_rows(r4, cr):
                    for dr in range(4):
                        r = r4 * 4 + dr
                        for j in range(4):
                            v = ebufs[b][pl.ds(r * PD + 16 * j, 16)]
                            fbufs[b][r, pl.ds(16 * j, 16)] = (
                                lax.bitcast_convert_type(v << 16,
                                                         jnp.float32))
                            fbufs[b][r, pl.ds(PD + 16 * j, 16)] = (
                                lax.bitcast_convert_type(
                                    v & jnp.int32(-65536), jnp.float32))
                    return cr

                lax.fori_loop(0, CHUNK // 4, conv_rows, 0)

                pltpu.async_copy(fbufs[b], acc.at[ibufs[b]], sss[b],
                                 add=True)

                nxt = i + NBUF

                @pl.when(nxt < NCHUNK)
                def _():
                    off = nxt * CHUNK
                    pltpu.async_copy(
                        comb_hbm.at[pl.ds((base + off) * PD, CHUNK * PD)],
                        ebufs[b], sgs[b])
                    pltpu.async_copy(idx_hbm.at[pl.ds(ibase + off, CHUNK)],
                                     tibufs[b], sgs[b])
            return carry

        lax.fori_loop(0, NCHUNK // NBUF, body, 0)

        for b in range(NBUF):
            pltpu.make_async_copy(fbufs[b], acc.at[ibufs[b]], sss[b]).wait()

        plsc.subcore_barrier()
        pltpu.sync_copy(acc.at[pl.ds(s * RPS, RPS)],
                        out_hbm.at[pl.ds(c * N + s * RPS, RPS)])

        @pl.when(s == 0)
        def _():
            pltpu.sync_copy(acc.at[pl.ds(NS * RPS, TAIL)],
                            out_hbm.at[pl.ds(c * N + NS * RPS, TAIL)])

    return run(comb, idx, init)


def _add_body(p_ref, q_ref, o_ref):
    o_ref[...] = p_ref[...] + q_ref[...]


def _final_add(partials):
    bn = 2000
    nblk = N // bn
    return pl.pallas_call(
        _add_body,
        grid=(nblk,),
        in_specs=[
            pl.BlockSpec((bn, D), lambda i: (i, 0)),
            pl.BlockSpec((bn, D), lambda i, _o=nblk: (i + _o, 0)),
        ],
        out_specs=pl.BlockSpec((bn, D), lambda i: (i, 0)),
        out_shape=jax.ShapeDtypeStruct((N, D), jnp.float32),
    )(partials, partials)


def kernel(x, index, dim, dim_size, W, learnable_param):
    del dim, dim_size
    wt = W.T
    idx = index.astype(jnp.int32)
    running = jnp.zeros((NC * N, D), jnp.float32)
    for k in range(NSEG):
        comb_k = _gate(x, wt, learnable_param, k).reshape(ESEG * PD)
        running = _sc_scatter(comb_k, idx, running, k)
    return _final_add(running)


# speedup vs baseline: 1.7804x; 1.7804x over previous
import functools

import jax
import jax.numpy as jnp
from jax import lax
from jax.experimental import pallas as pl
from jax.experimental.pallas import tpu as pltpu
from jax.experimental.pallas import tpu_sc as plsc

E = 320000
D = 128
PD = 64
N = 10000

NSEG = 2
ESEG = E // NSEG
NC = 2
NS = 16
NW = NC * NS
EPW = ESEG // NW
CHUNK = 40
NCHUNK = EPW // CHUNK
NBUF = 5
RPS = 624
TAIL = N - NS * RPS

GATE_B = 16000


def _gate_body(lp_ref, x_ref, wt_ref, out_ref):
    x = x_ref[...]
    a = x[:, :PD]
    b = x[:, PD:]
    wt = wt_ref[...]
    ga = jnp.clip(lax.dot(a, wt, preferred_element_type=jnp.float32), 0.0, 1.0)
    gb = jnp.clip(lax.dot(b, wt, preferred_element_type=jnp.float32), 0.0, 1.0)
    at = jnp.concatenate([a, a], axis=1)
    bt = jnp.concatenate([b, b], axis=1)
    lp = jnp.abs(lp_ref[0])
    out_ref[...] = lp * (at * ga + bt * gb) + x


def _gate(x, wt, lp, seg):
    nblk = ESEG // GATE_B
    return pl.pallas_call(
        _gate_body,
        grid=(nblk,),
        in_specs=[
            pl.BlockSpec(memory_space=pltpu.SMEM),
            pl.BlockSpec((GATE_B, D), lambda i, _o=seg * nblk: (i + _o, 0)),
            pl.BlockSpec((PD, D), lambda i: (0, 0)),
        ],
        out_specs=pl.BlockSpec((GATE_B, D), lambda i: (i, 0)),
        out_shape=jax.ShapeDtypeStruct((ESEG, D), jnp.float32),
    )(lp, x, wt)


def _sc_scatter(comb, idx, init, seg):
    mesh = plsc.VectorSubcoreMesh(core_axis_name="c", subcore_axis_name="s")
    ibase0 = seg * ESEG

    @functools.partial(
        pl.kernel,
        mesh=mesh,
        out_type=jax.ShapeDtypeStruct((NC * N, D), jnp.float32),
        scratch_types=(
            [pltpu.VMEM((CHUNK, D), jnp.float32) for _ in range(NBUF)]
            + [pltpu.VMEM((CHUNK,), jnp.int32) for _ in range(NBUF)]
            + [pltpu.VMEM_SHARED((N, D), jnp.float32)]
            + [pltpu.SemaphoreType.DMA for _ in range(NBUF)]
        ),
    )
    def run(comb_hbm, idx_hbm, init_hbm, out_hbm,
            eb0, eb1, eb2, eb3, eb4, ib0, ib1, ib2, ib3, ib4,
            acc, sg0, sg1, sg2, sg3, sg4):
        ebufs = [eb0, eb1, eb2, eb3, eb4]
        ibufs = [ib0, ib1, ib2, ib3, ib4]
        sgs = [sg0, sg1, sg2, sg3, sg4]
        c = lax.axis_index("c")
        s = lax.axis_index("s")
        wid = s * NC + c
        base = wid * EPW
        ibase = ibase0 + base

        for b in range(NBUF):
            off = b * CHUNK
            pltpu.async_copy(comb_hbm.at[pl.ds(base + off, CHUNK)],
                             ebufs[b], sgs[b])
            pltpu.async_copy(idx_hbm.at[pl.ds(ibase + off, CHUNK)],
                             ibufs[b], sgs[b])

        pltpu.sync_copy(init_hbm.at[pl.ds(c * N + s * RPS, RPS)],
                        acc.at[pl.ds(s * RPS, RPS)])

        @pl.when(s == 0)
        def _():
            pltpu.sync_copy(init_hbm.at[pl.ds(c * N + NS * RPS, TAIL)],
                            acc.at[pl.ds(NS * RPS, TAIL)])

        plsc.subcore_barrier()

        def body(g, carry):
            for b in range(NBUF):
                i = g * NBUF + b
                pltpu.make_async_copy(
                    comb_hbm.at[pl.ds(base, CHUNK)], ebufs[b], sgs[b]).wait()
                pltpu.make_async_copy(
                    idx_hbm.at[pl.ds(base, CHUNK)], ibufs[b], sgs[b]).wait()
                pltpu.sync_copy(ebufs[b], acc.at[ibufs[b]], add=True)
                nxt = i + NBUF

                @pl.when(nxt < NCHUNK)
                def _():
                    off = nxt * CHUNK
                    pltpu.async_copy(comb_hbm.at[pl.ds(base + off, CHUNK)],
                                     ebufs[b], sgs[b])
                    pltpu.async_copy(idx_hbm.at[pl.ds(ibase + off, CHUNK)],
                                     ibufs[b], sgs[b])
            return carry

        lax.fori_loop(0, NCHUNK // NBUF, body, 0)
        plsc.subcore_barrier()
        pltpu.sync_copy(acc.at[pl.ds(s * RPS, RPS)],
                        out_hbm.at[pl.ds(c * N + s * RPS, RPS)])

        @pl.when(s == 0)
        def _():
            pltpu.sync_copy(acc.at[pl.ds(NS * RPS, TAIL)],
                            out_hbm.at[pl.ds(c * N + NS * RPS, TAIL)])

    return run(comb, idx, init)


def _add_body(p_ref, q_ref, o_ref):
    o_ref[...] = p_ref[...] + q_ref[...]


def _final_add(partials):
    bn = 2000
    nblk = N // bn
    return pl.pallas_call(
        _add_body,
        grid=(nblk,),
        in_specs=[
            pl.BlockSpec((bn, D), lambda i: (i, 0)),
            pl.BlockSpec((bn, D), lambda i, _o=nblk: (i + _o, 0)),
        ],
        out_specs=pl.BlockSpec((bn, D), lambda i: (i, 0)),
        out_shape=jax.ShapeDtypeStruct((N, D), jnp.float32),
    )(partials, partials)


def kernel(x, index, dim, dim_size, W, learnable_param):
    del dim, dim_size
    wt = W.T
    idx = index.astype(jnp.int32)
    running = jnp.zeros((NC * N, D), jnp.float32)
    for k in range(NSEG):
        comb_k = _gate(x, wt, learnable_param, k)
        running = _sc_scatter(comb_k, idx, running, k)
    return _final_add(running)
